# baseline (speedup 1.0000x reference)
import functools

import jax
import jax.numpy as jnp
from jax import lax
from jax.experimental import pallas as pl
from jax.experimental.pallas import tpu as pltpu
from jax.experimental.pallas import tpu_sc as plsc

VOCAB = 1000000
SEQLEN = 200
EMBED = 64
BATCH = 4096
LANES = 16

ROWS = BATCH * SEQLEN
NW = 32
RPW = ROWS // NW
TILE = 400
NT = RPW // TILE
GCH = 80
NG = TILE // GCH
STS = 8
NST = NT // STS


def _make_kernel():
    mesh = plsc.VectorSubcoreMesh(core_axis_name="c", subcore_axis_name="s")

    @functools.partial(
        pl.kernel,
        mesh=mesh,
        out_type=jax.ShapeDtypeStruct((ROWS, EMBED), jnp.float32),
        compiler_params=pltpu.CompilerParams(use_tc_tiling_on_sc=False),
        scratch_types=[
            pltpu.VMEM((STS * NG, GCH), jnp.int32),
            pltpu.VMEM((TILE, EMBED), jnp.float32),
            pltpu.VMEM((TILE, EMBED), jnp.float32),
            pltpu.SemaphoreType.DMA,
        ],
    )
    def emb(table_hbm, idx_hbm, pos_hbm, out_hbm, idx_v, rows_v, pos_v, sem):
        wid = lax.axis_index("s") * 2 + lax.axis_index("c")
        base = wid * RPW
        idx_row0 = base // GCH

        pltpu.sync_copy(pos_hbm, pos_v.at[pl.ds(0, SEQLEN)])
        pltpu.sync_copy(pos_hbm, pos_v.at[pl.ds(SEQLEN, SEQLEN)])

        def super_body(st, carry):
            pltpu.sync_copy(
                idx_hbm.at[
                    pl.ds(
                        pl.multiple_of(idx_row0 + st * (STS * NG), 8), STS * NG
                    )
                ],
                idx_v,
            )

            def tile_body(t8, carry2):
                copies = [
                    pltpu.async_copy(
                        table_hbm.at[idx_v.at[t8 * NG + g]],
                        rows_v.at[pl.ds(g * GCH, GCH)],
                        sem,
                    )
                    for g in range(NG)
                ]
                for c in copies:
                    c.wait()

                def add_body(r, _):
                    for q in range(EMBED // LANES):
                        sl = pl.ds(q * LANES, LANES)
                        rows_v[r, sl] = rows_v[r, sl] + pos_v[r, sl]
                    return 0

                lax.fori_loop(0, TILE, add_body, 0, unroll=2)

                pltpu.sync_copy(
                    rows_v,
                    out_hbm.at[
                        pl.ds(
                            pl.multiple_of(
                                base + (st * STS + t8) * TILE, 8
                            ),
                            TILE,
                        )
                    ],
                )
                return carry2

            lax.fori_loop(0, STS, tile_body, 0)
            return carry

        lax.fori_loop(0, NST, super_body, 0)

    return emb


_emb = _make_kernel()


@jax.jit
def kernel(inp, token_table, pos_table):
    idx = inp.astype(jnp.int32).reshape(ROWS // GCH, GCH)
    out = _emb(token_table, idx, pos_table)
    return out.reshape(BATCH, SEQLEN, EMBED)


# speedup vs baseline: 1.0748x; 1.0748x over previous
import functools

import jax
import jax.numpy as jnp
from jax import lax
from jax.experimental import pallas as pl
from jax.experimental.pallas import tpu as pltpu
from jax.experimental.pallas import tpu_sc as plsc

VOCAB = 1000000
SEQLEN = 200
EMBED = 64
BATCH = 4096
LANES = 16

ROWS = BATCH * SEQLEN
NW = 32
RPW = ROWS // NW
TILE = 400
NT = RPW // TILE
GCH = 80
NG = TILE // GCH


def _make_kernel():
    mesh = plsc.VectorSubcoreMesh(core_axis_name="c", subcore_axis_name="s")

    @functools.partial(
        pl.kernel,
        mesh=mesh,
        out_type=jax.ShapeDtypeStruct((ROWS, EMBED), jnp.float32),
        compiler_params=pltpu.CompilerParams(use_tc_tiling_on_sc=False),
        scratch_types=[
            pltpu.VMEM((RPW // GCH, GCH), jnp.int32),
            pltpu.VMEM((TILE, EMBED), jnp.float32),
            pltpu.VMEM((TILE, EMBED), jnp.float32),
            pltpu.VMEM((TILE, EMBED), jnp.float32),
            pltpu.SemaphoreType.DMA,
            pltpu.SemaphoreType.DMA,
            pltpu.SemaphoreType.DMA,
            pltpu.SemaphoreType.DMA,
        ],
    )
    def emb(
        table_hbm, idx_hbm, pos_hbm, out_hbm,
        idx_v, rows0, rows1, pos_v, gsem0, gsem1, wsem0, wsem1,
    ):
        wid = lax.axis_index("s") * 2 + lax.axis_index("c")
        base = wid * RPW
        idx_row0 = base // GCH

        rows = (rows0, rows1)
        gsem = (gsem0, gsem1)
        wsem = (wsem0, wsem1)

        pltpu.sync_copy(
            idx_hbm.at[pl.ds(pl.multiple_of(idx_row0, 8), RPW // GCH)], idx_v
        )
        pltpu.sync_copy(pos_hbm, pos_v.at[pl.ds(0, SEQLEN)])
        pltpu.sync_copy(pos_hbm, pos_v.at[pl.ds(SEQLEN, SEQLEN)])

        def gather_copies(t, b):
            return [
                pltpu.make_async_copy(
                    table_hbm.at[idx_v.at[t * NG + g]],
                    rows[b].at[pl.ds(g * GCH, GCH)],
                    gsem[b],
                )
                for g in range(NG)
            ]

        def write_copy(t, b):
            return pltpu.make_async_copy(
                rows[b],
                out_hbm.at[pl.ds(pl.multiple_of(base + t * TILE, 8), TILE)],
                wsem[b],
            )

        def add_pos(b):
            def add_body(r, _):
                for q in range(EMBED // LANES):
                    sl = pl.ds(q * LANES, LANES)
                    rows[b][r, sl] = rows[b][r, sl] + pos_v[r, sl]
                return 0

            lax.fori_loop(0, TILE, add_body, 0, unroll=2)

        for c in gather_copies(0, 0):
            c.start()

        def loop_body(i, carry):
            g = i * 2
            @pl.when(g > 0)
            def _():
                write_copy(g - 1, 1).wait()
            for c in gather_copies(g + 1, 1):
                c.start()
            for c in gather_copies(g, 0):
                c.wait()
            add_pos(0)
            write_copy(g, 0).start()

            write_copy(g, 0).wait()

            @pl.when(g < NT - 2)
            def _():
                for c in gather_copies(g + 2, 0):
                    c.start()

            for c in gather_copies(g + 1, 1):
                c.wait()
            add_pos(1)
            write_copy(g + 1, 1).start()
            return carry

        lax.fori_loop(0, NT // 2, loop_body, 0, unroll=False)

        write_copy(NT - 1, 1).wait()

    return emb


_emb = _make_kernel()


@jax.jit
def kernel(inp, token_table, pos_table):
    idx = inp.astype(jnp.int32).reshape(ROWS // GCH, GCH)
    out = _emb(token_table, idx, pos_table)
    return out.reshape(BATCH, SEQLEN, EMBED)
